# Initial kernel scaffold; baseline (speedup 1.0000x reference)
import jax
import jax.numpy as jnp
from jax.experimental import pallas as pl


def kernel(q, k, rel_pos_enc):
    raise NotImplementedError("write your pallas kernel here")


# speedup vs baseline: 11.7849x; 11.7849x over previous
import functools

import jax
import jax.numpy as jnp
from jax.experimental import pallas as pl

MAX_LEN = 512
BLOCK_ROWS = 8


def _copy_kernel(t8_ref, out_ref, *, block_rows, seq_len_k, max_len):
    pid = pl.program_id(0)
    for r in range(block_rows):
        i = pid * block_rows + r
        s = (max_len - 1) - i
        c = jax.lax.rem(s, 8)
        aligned = pl.multiple_of(s - c, 8)
        out_ref[r] = t8_ref[c, pl.ds(aligned, seq_len_k), :]


def kernel(q, k, rel_pos_enc):
    seq_len_q = q.shape[1]
    seq_len_k = k.shape[1]
    d = rel_pos_enc.shape[1]
    n = rel_pos_enc.shape[0]

    n_pad = ((n + 7) // 8) * 8 + 8
    padded = jnp.pad(rel_pos_enc, ((0, n_pad + 8 - n), (0, 0)))
    t8 = jnp.stack([jax.lax.dynamic_slice_in_dim(padded, c, n_pad, 0)
                    for c in range(8)])

    br = BLOCK_ROWS
    grid = (seq_len_q // br,)
    body = functools.partial(
        _copy_kernel, block_rows=br, seq_len_k=seq_len_k, max_len=MAX_LEN
    )
    return pl.pallas_call(
        body,
        grid=grid,
        in_specs=[
            pl.BlockSpec(t8.shape, lambda i: (0, 0, 0)),
        ],
        out_specs=pl.BlockSpec((br, seq_len_k, d), lambda i: (i, 0, 0)),
        out_shape=jax.ShapeDtypeStruct((seq_len_q, seq_len_k, d), rel_pos_enc.dtype),
    )(t8)
